# Initial kernel scaffold; baseline (speedup 1.0000x reference)
import jax
import jax.numpy as jnp
from jax.experimental import pallas as pl


def kernel(obs, action, W, b, gamma, beta, keys1, values1, keys2, values2):
    raise NotImplementedError("write your pallas kernel here")


# speedup vs baseline: 7.5592x; 7.5592x over previous
import functools

import jax
import jax.numpy as jnp
from jax import lax
from jax.experimental import pallas as pl
from jax.experimental.pallas import tpu as pltpu
from jax.experimental.pallas import tpu_sc as plsc

B = 1024
OBS_DIM = 256
ACT_DIM = 64
HID = 512
CAP = 65536
K = 32
DELTA = 1e-3
KEY_BLK = 2048
N_KBLK = CAP // KEY_BLK
CHUNK = 128
N_CHUNK = CAP // CHUNK
CHUNKS_PER_BLK = KEY_BLK // CHUNK
CAND = K * CHUNK
ROW_BLK = 128
_PREC = lax.Precision.HIGHEST


def _trunk_kernel(obs_ref, act_ref, w_ref, b_ref, g_ref, be_ref, phi_ref):
    bf = jnp.bfloat16
    x = (lax.dot_general(obs_ref[...].astype(bf), w_ref[:OBS_DIM, :].astype(bf),
                         (((1,), (0,)), ((), ())),
                         preferred_element_type=jnp.float32)
         + lax.dot_general(act_ref[...].astype(bf), w_ref[OBS_DIM:, :].astype(bf),
                           (((1,), (0,)), ((), ())),
                           preferred_element_type=jnp.float32)
         + b_ref[...])
    mu = jnp.mean(x, axis=-1, keepdims=True)
    var = jnp.mean((x - mu) ** 2, axis=-1, keepdims=True)
    x = (x - mu) / jnp.sqrt(var + 1e-5) * g_ref[...] + be_ref[...]
    phi_ref[...] = jnp.tanh(x)


def _dist_kernel(phi_ref, keys_ref, d_ref, m_ref):
    phi = phi_ref[...]
    kb = keys_ref[...]
    pn = jnp.sum(phi * phi, axis=1, keepdims=True)
    kn = jnp.sum(kb * kb, axis=1, keepdims=True)
    g = lax.dot_general(phi.astype(jnp.bfloat16), kb.astype(jnp.bfloat16),
                        (((1,), (1,)), ((), ())),
                        preferred_element_type=jnp.float32)
    d = jnp.maximum(pn - 2.0 * g + kn.T, 0.0)
    d_ref[...] = d
    m_ref[...] = jnp.min(
        d.reshape(B, CHUNKS_PER_BLK, CHUNK), axis=-1)[None]


def _select_kernel(m_ref, g1_ref, g2_ref):
    pid = pl.program_id(0)
    mv = m_ref[...]
    iota = lax.broadcasted_iota(jnp.int32, (ROW_BLK, N_CHUNK), 1)
    big = jnp.int32(1 << 24)
    cols = []
    for _ in range(K):
        m = jnp.min(mv, axis=1, keepdims=True)
        am = jnp.min(jnp.where(mv == m, iota, big), axis=1, keepdims=True)
        mv = jnp.where(iota == am, jnp.inf, mv)
        cols.append(am)
    c32 = jnp.concatenate(cols, axis=1)
    rows = (pid * ROW_BLK
            + lax.broadcasted_iota(jnp.int32, (ROW_BLK, K), 0))
    g1_ref[...] = rows * N_CHUNK + c32
    g2_ref[...] = c32


def _final_kernel(c_ref, v_ref, q_ref):
    c = c_ref[...]
    v = v_ref[...]
    iota = lax.broadcasted_iota(jnp.int32, (ROW_BLK, CAND), 1)
    big = jnp.int32(1 << 24)
    sw = jnp.zeros((ROW_BLK, 1), jnp.float32)
    swv = jnp.zeros((ROW_BLK, 1), jnp.float32)
    for _ in range(K):
        m = jnp.min(c, axis=1, keepdims=True)
        am = jnp.min(jnp.where(c == m, iota, big), axis=1, keepdims=True)
        sel = iota == am
        vm = jnp.sum(jnp.where(sel, v, 0.0), axis=1, keepdims=True)
        w = 1.0 / (m + DELTA)
        sw += w
        swv += w * vm
        c = jnp.where(sel, jnp.inf, c)
    q_ref[...] = swv / sw


def _sc_gather(dc, vc, g1, g2):
    info = plsc.get_sparse_core_info()
    nw = info.num_cores * info.num_subcores
    total = B * K
    per_w = total // nw
    ch = 256
    n_ch = per_w // ch
    mesh = plsc.VectorSubcoreMesh(core_axis_name="c", subcore_axis_name="s")
    out_sd = jax.ShapeDtypeStruct((total, CHUNK), jnp.float32)

    @functools.partial(
        pl.kernel, mesh=mesh, out_type=(out_sd, out_sd),
        scratch_types=[
            pltpu.VMEM((ch,), jnp.int32),
            pltpu.VMEM((ch, CHUNK), jnp.float32),
            pltpu.SemaphoreType.DMA,
        ],
    )
    def k(dc_hbm, vc_hbm, g1_hbm, g2_hbm, cand_hbm, vcand_hbm,
          idx_v, rows_v, sem):
        wid = lax.axis_index("s") * info.num_cores + lax.axis_index("c")
        for c in range(n_ch):
            base = wid * per_w + c * ch
            pltpu.sync_copy(g1_hbm.at[pl.ds(base, ch)], idx_v)
            pltpu.async_copy(dc_hbm.at[idx_v], rows_v, sem).wait()
            pltpu.sync_copy(rows_v, cand_hbm.at[pl.ds(base, ch)])
            pltpu.sync_copy(g2_hbm.at[pl.ds(base, ch)], idx_v)
            pltpu.async_copy(vc_hbm.at[idx_v], rows_v, sem).wait()
            pltpu.sync_copy(rows_v, vcand_hbm.at[pl.ds(base, ch)])

    return k(dc, vc, g1, g2)


def kernel(obs, action, W, b, gamma, beta, keys1, values1, keys2, values2):
    f32 = jnp.float32
    phi = pl.pallas_call(
        _trunk_kernel,
        out_shape=jax.ShapeDtypeStruct((B, HID), f32),
    )(obs, action, W, b.reshape(1, HID), gamma.reshape(1, HID),
      beta.reshape(1, HID))

    d, m3 = pl.pallas_call(
        _dist_kernel,
        grid=(N_KBLK,),
        in_specs=[
            pl.BlockSpec((B, HID), lambda j: (0, 0)),
            pl.BlockSpec((KEY_BLK, HID), lambda j: (j, 0)),
        ],
        out_specs=[
            pl.BlockSpec((B, KEY_BLK), lambda j: (0, j)),
            pl.BlockSpec((1, B, CHUNKS_PER_BLK), lambda j: (j, 0, 0)),
        ],
        out_shape=[
            jax.ShapeDtypeStruct((B, CAP), f32),
            jax.ShapeDtypeStruct((N_KBLK, B, CHUNKS_PER_BLK), f32),
        ],
    )(phi, keys1)

    m = jnp.transpose(m3, (1, 0, 2)).reshape(B, N_CHUNK)

    g1, g2 = pl.pallas_call(
        _select_kernel,
        grid=(B // ROW_BLK,),
        in_specs=[pl.BlockSpec((ROW_BLK, N_CHUNK), lambda i: (i, 0))],
        out_specs=[
            pl.BlockSpec((ROW_BLK, K), lambda i: (i, 0)),
            pl.BlockSpec((ROW_BLK, K), lambda i: (i, 0)),
        ],
        out_shape=[
            jax.ShapeDtypeStruct((B, K), jnp.int32),
            jax.ShapeDtypeStruct((B, K), jnp.int32),
        ],
    )(m)

    cand, vcand = _sc_gather(
        d.reshape(B * N_CHUNK, CHUNK),
        values1.reshape(N_CHUNK, CHUNK),
        g1.reshape(B * K),
        g2.reshape(B * K),
    )

    q = pl.pallas_call(
        _final_kernel,
        grid=(B // ROW_BLK,),
        in_specs=[
            pl.BlockSpec((ROW_BLK, CAND), lambda i: (i, 0)),
            pl.BlockSpec((ROW_BLK, CAND), lambda i: (i, 0)),
        ],
        out_specs=pl.BlockSpec((ROW_BLK, 1), lambda i: (i, 0)),
        out_shape=jax.ShapeDtypeStruct((B, 1), f32),
    )(cand.reshape(B, CAND), vcand.reshape(B, CAND))

    return (q, q)
